# Initial kernel scaffold; baseline (speedup 1.0000x reference)
import jax
import jax.numpy as jnp
from jax.experimental import pallas as pl


def kernel(z, batch_ids, emb, W1, b1, W2, b2):
    raise NotImplementedError("write your pallas kernel here")


# speedup vs baseline: 18.7023x; 18.7023x over previous
import functools

import jax
import jax.numpy as jnp
from jax import lax
from jax.experimental import pallas as pl
from jax.experimental.pallas import tpu as pltpu
from jax.experimental.pallas import tpu_sc as plsc

N = 100_000
G = 2048
H = 128
VP = 128
CHUNK = 2048
NW = 32
GPW = G // NW
NP = -(-N // CHUNK) * CHUNK
BBUF = 48


def _bounds_body(b_ref, out_ref):
    b = b_ref[...]
    out_ref[0] = jnp.int32(0)
    for t in range(1, NW + 1):
        out_ref[t] = jnp.sum((b < t * GPW).astype(jnp.int32))
    for t in range(NW + 1, BBUF):
        out_ref[t] = jnp.int32(0)


def _bounds_call(b2d):
    return pl.pallas_call(
        _bounds_body,
        out_shape=jax.ShapeDtypeStruct((BBUF,), jnp.int32),
        in_specs=[pl.BlockSpec(memory_space=pltpu.VMEM)],
        out_specs=pl.BlockSpec(memory_space=pltpu.SMEM),
    )(b2d)


_sc_mesh = plsc.VectorSubcoreMesh(core_axis_name="c", subcore_axis_name="s")


@functools.partial(
    pl.kernel,
    mesh=_sc_mesh,
    out_type=jax.ShapeDtypeStruct((G * VP,), jnp.float32),
    scratch_types=[
        pltpu.VMEM((CHUNK,), jnp.int32),
        pltpu.VMEM((CHUNK,), jnp.int32),
        pltpu.VMEM((GPW * VP,), jnp.float32),
        pltpu.VMEM((BBUF,), jnp.int32),
    ],
    compiler_params=pltpu.CompilerParams(needs_layout_passes=False),
)
def _sc_hist(z_hbm, b_hbm, bounds_hbm, out_hbm, zbuf, bbuf, hist, bnd):
    wid = lax.axis_index("s") * 2 + lax.axis_index("c")
    pltpu.sync_copy(bounds_hbm, bnd)

    lo = bnd[pl.ds(wid, 16)][0]
    hi = bnd[pl.ds(wid + 1, 16)][0]

    zeros16 = jnp.zeros((16,), jnp.float32)
    ones16 = jnp.ones((16,), jnp.float32)

    def zero_body(i, carry):
        hist[pl.ds(i * 16, 16)] = zeros16
        return carry

    lax.fori_loop(0, (GPW * VP) // 16, zero_body, 0)

    g_base = wid * GPW
    c0 = lo // CHUNK
    c1 = (hi + CHUNK - 1) // CHUNK

    def chunk_body(c, carry):
        pltpu.sync_copy(z_hbm.at[pl.ds(c * CHUNK, CHUNK)], zbuf)
        pltpu.sync_copy(b_hbm.at[pl.ds(c * CHUNK, CHUNK)], bbuf)

        def vec_body(i, inner):
            zv = zbuf[pl.ds(i * 16, 16)]
            bv = bbuf[pl.ds(i * 16, 16)]
            rel = bv - g_base
            msk = (rel >= 0) & (rel < GPW)
            flat = jnp.where(msk, rel * VP + zv, 0)
            plsc.addupdate_scatter(hist, [flat], ones16, mask=msk)
            return inner

        lax.fori_loop(0, CHUNK // 16, vec_body, 0)
        return carry

    lax.fori_loop(c0, c1, chunk_body, 0)

    pltpu.sync_copy(hist, out_hbm.at[pl.ds(g_base * VP, GPW * VP)])


def _head_body(hist_ref, emb_ref, w1t_ref, b1_ref, w2t_ref, b2_ref, out_ref):
    hist = hist_ref[...]
    counts = jnp.sum(hist, axis=1, keepdims=True)
    denom = jnp.maximum(counts, 1.0)
    sums = lax.dot_general(hist, emb_ref[...], (((1,), (0,)), ((), ())),
                           preferred_element_type=jnp.float32)
    pooled = sums / denom
    h = jnp.maximum(
        lax.dot_general(pooled, w1t_ref[...], (((1,), (0,)), ((), ())),
                        preferred_element_type=jnp.float32) + b1_ref[...],
        0.0)
    out = (lax.dot_general(h, w2t_ref[...], (((1,), (0,)), ((), ())),
                           preferred_element_type=jnp.float32) + b2_ref[0])
    out_ref[...] = out


def _head_call(hist, emb_pad, w1t, b1, w2t, b2):
    return pl.pallas_call(
        _head_body,
        out_shape=jax.ShapeDtypeStruct((G, 1), jnp.float32),
        in_specs=[
            pl.BlockSpec(memory_space=pltpu.VMEM),
            pl.BlockSpec(memory_space=pltpu.VMEM),
            pl.BlockSpec(memory_space=pltpu.VMEM),
            pl.BlockSpec(memory_space=pltpu.VMEM),
            pl.BlockSpec(memory_space=pltpu.VMEM),
            pl.BlockSpec(memory_space=pltpu.SMEM),
        ],
        out_specs=pl.BlockSpec(memory_space=pltpu.VMEM),
    )(hist, emb_pad, w1t, b1, w2t, b2)


def kernel(z, batch_ids, emb, W1, b1, W2, b2):
    z = z.astype(jnp.int32)
    b = batch_ids.astype(jnp.int32)
    pad = NP - N
    z_pad = jnp.concatenate([z, jnp.zeros((pad,), jnp.int32)])
    b_pad = jnp.concatenate([b, jnp.full((pad,), G, jnp.int32)])

    bounds = _bounds_call(b_pad.reshape(NP // 128, 128))
    hist = _sc_hist(z_pad, b_pad, bounds).reshape(G, VP)

    V = emb.shape[0]
    emb_pad = jnp.zeros((VP, H), jnp.float32).at[:V].set(emb.astype(jnp.float32))
    out = _head_call(
        hist,
        emb_pad,
        W1.astype(jnp.float32).T,
        b1.astype(jnp.float32).reshape(1, H),
        W2.astype(jnp.float32).T,
        b2.astype(jnp.float32),
    )
    return out


# speedup vs baseline: 21.0733x; 1.1268x over previous
import functools

import jax
import jax.numpy as jnp
from jax import lax
from jax.experimental import pallas as pl
from jax.experimental.pallas import tpu as pltpu
from jax.experimental.pallas import tpu_sc as plsc

N = 100_000
G = 2048
H = 128
VP = 128
CHUNK = 2048
NW = 32
GPW = G // NW
BBUF = 48


def _bounds_body(b_ref, out_ref):
    b = b_ref[...]
    out_ref[0] = jnp.int32(0)
    for t in range(1, NW):
        out_ref[t] = jnp.sum((b < t * GPW).astype(jnp.int32))
    out_ref[NW] = jnp.int32(N)
    for t in range(NW + 1, BBUF):
        out_ref[t] = jnp.int32(0)


def _bounds_call(b2d):
    return pl.pallas_call(
        _bounds_body,
        out_shape=jax.ShapeDtypeStruct((BBUF,), jnp.int32),
        in_specs=[pl.BlockSpec(memory_space=pltpu.VMEM)],
        out_specs=pl.BlockSpec(memory_space=pltpu.SMEM),
    )(b2d)


_sc_mesh = plsc.VectorSubcoreMesh(core_axis_name="c", subcore_axis_name="s")


@functools.partial(
    pl.kernel,
    mesh=_sc_mesh,
    out_type=jax.ShapeDtypeStruct((G * VP,), jnp.float32),
    scratch_types=[
        pltpu.VMEM((CHUNK,), jnp.int32),
        pltpu.VMEM((CHUNK,), jnp.int32),
        pltpu.VMEM((GPW * VP,), jnp.float32),
        pltpu.VMEM((BBUF,), jnp.int32),
        pltpu.SemaphoreType.DMA,
        pltpu.SemaphoreType.DMA,
    ],
    compiler_params=pltpu.CompilerParams(needs_layout_passes=False),
)
def _sc_hist(z_hbm, b_hbm, bounds_hbm, out_hbm, zbuf, bbuf, hist, bnd,
             sem0, sem1):
    wid = lax.axis_index("s") * 2 + lax.axis_index("c")
    pltpu.sync_copy(bounds_hbm, bnd)

    lo = bnd[pl.ds(wid, 16)][0]
    hi = bnd[pl.ds(wid + 1, 16)][0]

    zeros16 = jnp.zeros((16,), jnp.float32)
    ones16 = jnp.ones((16,), jnp.float32)
    lanes = jnp.arange(16, dtype=jnp.int32)

    def zero_body(i, carry):
        hist[pl.ds(i * 16, 16)] = zeros16
        return carry

    lax.fori_loop(0, (GPW * VP) // 16, zero_body, 0, unroll=8)

    g_base = wid * GPW
    c0 = lo // CHUNK
    c1 = (hi + CHUNK - 1) // CHUNK

    def chunk_body(c, carry):
        start = c * CHUNK
        base = jnp.minimum(start, N - CHUNK)
        cz = pltpu.async_copy(z_hbm.at[pl.ds(base, CHUNK)], zbuf, sem0)
        cb = pltpu.async_copy(b_hbm.at[pl.ds(base, CHUNK)], bbuf, sem1)
        cz.wait()
        cb.wait()
        skip = start - base

        def vec_body(i, inner):
            zv = zbuf[pl.ds(i * 16, 16)]
            bv = bbuf[pl.ds(i * 16, 16)]
            rel = bv - g_base
            msk = (rel >= 0) & (rel < GPW) & (i * 16 + lanes >= skip)
            flat = jnp.where(msk, rel * VP + zv, 0)
            plsc.addupdate_scatter(hist, [flat], ones16, mask=msk)
            return inner

        lax.fori_loop(0, CHUNK // 16, vec_body, 0, unroll=4)
        return carry

    lax.fori_loop(c0, c1, chunk_body, 0)

    pltpu.sync_copy(hist, out_hbm.at[pl.ds(g_base * VP, GPW * VP)])


def _head_body(hist_ref, emb_ref, w1_ref, b1_ref, w2_ref, b2_ref, out_ref):
    hist = hist_ref[...]
    counts = jnp.sum(hist, axis=1, keepdims=True)
    denom = jnp.maximum(counts, 1.0)
    sums = lax.dot_general(hist[:, :100], emb_ref[...],
                           (((1,), (0,)), ((), ())),
                           preferred_element_type=jnp.float32)
    pooled = sums / denom
    h = jnp.maximum(
        lax.dot_general(pooled, w1_ref[...], (((1,), (1,)), ((), ())),
                        preferred_element_type=jnp.float32) + b1_ref[...],
        0.0)
    out = (lax.dot_general(h, w2_ref[...], (((1,), (0,)), ((), ())),
                           preferred_element_type=jnp.float32) + b2_ref[0])
    out_ref[...] = out


def _head_call(hist, emb, w1, b1, w2, b2):
    return pl.pallas_call(
        _head_body,
        out_shape=jax.ShapeDtypeStruct((G, 1), jnp.float32),
        in_specs=[
            pl.BlockSpec(memory_space=pltpu.VMEM),
            pl.BlockSpec(memory_space=pltpu.VMEM),
            pl.BlockSpec(memory_space=pltpu.VMEM),
            pl.BlockSpec(memory_space=pltpu.VMEM),
            pl.BlockSpec(memory_space=pltpu.VMEM),
            pl.BlockSpec(memory_space=pltpu.SMEM),
        ],
        out_specs=pl.BlockSpec(memory_space=pltpu.VMEM),
    )(hist, emb, w1, b1, w2, b2)


def kernel(z, batch_ids, emb, W1, b1, W2, b2):
    z = z.astype(jnp.int32)
    b = batch_ids.astype(jnp.int32)

    bounds = _bounds_call(b.reshape(100, 1000))
    hist = _sc_hist(z, b, bounds).reshape(G, VP)

    out = _head_call(
        hist,
        emb.astype(jnp.float32),
        W1.astype(jnp.float32),
        b1.astype(jnp.float32).reshape(1, H),
        W2.astype(jnp.float32).T,
        b2.astype(jnp.float32),
    )
    return out
